# baseline (speedup 1.0000x reference)
import functools

import jax
import jax.numpy as jnp
from jax import lax
from jax.experimental import pallas as pl
from jax.experimental.pallas import tpu as pltpu
from jax.experimental.pallas import tpu_sc as plsc

NC = 2
NS = 16
L = 16
NW = NC * NS

CH = 80


def _sc_edge_pass(E, N, H, with_type_and_degrees, bf16_table=False):
  EPW = E // NW
  NCH = EPW // CH
  NPT = N // NS
  ZR = 25
  DB = 1000
  ND = N // DB
  assert E % NW == 0 and EPW % CH == 0 and N % NS == 0 and NPT % ZR == 0
  assert N % DB == 0

  mesh = plsc.VectorSubcoreMesh(core_axis_name="c", subcore_axis_name="s")

  out_type = [jax.ShapeDtypeStruct((NC, NS, NPT, H), jnp.float32)]
  if with_type_and_degrees:
    out_type += [jax.ShapeDtypeStruct((ND, NW, 1, DB), jnp.float32),
                 jax.ShapeDtypeStruct((ND, NW, 1, DB), jnp.float32)]

  scratch = dict(
      u_t=pltpu.VMEM((EPW,), jnp.int32),
      v_t=pltpu.VMEM((EPW,), jnp.int32),
      norm_t=pltpu.VMEM((EPW,), jnp.float32),
      idxg0=pltpu.VMEM((CH,), jnp.int32),
      idxg1=pltpu.VMEM((CH,), jnp.int32),
      idxg2=pltpu.VMEM((CH,), jnp.int32),
      idxs0=pltpu.VMEM((CH,), jnp.int32),
      idxs1=pltpu.VMEM((CH,), jnp.int32),
      idxs2=pltpu.VMEM((CH,), jnp.int32),
      rows0=pltpu.VMEM((CH, H), jnp.float32),
      rows1=pltpu.VMEM((CH, H), jnp.float32),
      rows2=pltpu.VMEM((CH, H), jnp.float32),
      zbuf=pltpu.VMEM((ZR, H), jnp.float32),
      **({"rbf0": pltpu.VMEM((CH, H), jnp.bfloat16),
          "rbf1": pltpu.VMEM((CH, H), jnp.bfloat16),
          "rbf2": pltpu.VMEM((CH, H), jnp.bfloat16)} if bf16_table else {}),
      agg_sh=pltpu.VMEM_SHARED((N, H), jnp.float32),
      gsem0=pltpu.SemaphoreType.DMA,
      gsem1=pltpu.SemaphoreType.DMA,
      gsem2=pltpu.SemaphoreType.DMA,
      ssem0=pltpu.SemaphoreType.DMA,
      ssem1=pltpu.SemaphoreType.DMA,
      ssem2=pltpu.SemaphoreType.DMA,
  )
  if with_type_and_degrees:
    scratch.update(
        t_t=pltpu.VMEM((EPW,), jnp.int32),
        odl=pltpu.VMEM((N,), jnp.float32),
        idl=pltpu.VMEM((N,), jnp.float32),
    )

  names = list(scratch.keys())
  types = [scratch[k] for k in names]

  def body(*refs):
    n_in = 5 if with_type_and_degrees else 4
    n_out = len(out_type)
    ins = refs[:n_in]
    outs = refs[n_in:n_in + n_out]
    scr = dict(zip(names, refs[n_in + n_out:]))
    if with_type_and_degrees:
      u_hbm, v_hbm, t_hbm, norm_hbm, tab_hbm = ins
      agg_out, od_out, id_out = outs
    else:
      u_hbm, v_hbm, norm_hbm, tab_hbm = ins
      (agg_out,) = outs
    u_t, v_t, norm_t = scr["u_t"], scr["v_t"], scr["norm_t"]
    idxg = [scr["idxg0"], scr["idxg1"], scr["idxg2"]]
    idxs = [scr["idxs0"], scr["idxs1"], scr["idxs2"]]
    rows = [scr["rows0"], scr["rows1"], scr["rows2"]]
    gsem = [scr["gsem0"], scr["gsem1"], scr["gsem2"]]
    ssem = [scr["ssem0"], scr["ssem1"], scr["ssem2"]]
    gdst = ([scr["rbf0"], scr["rbf1"], scr["rbf2"]] if bf16_table else rows)
    zbuf, agg_sh = scr["zbuf"], scr["agg_sh"]

    cid = lax.axis_index("c")
    sid = lax.axis_index("s")
    wid = sid * NC + cid
    ebase = wid * EPW

    zero16 = jnp.zeros((L,), jnp.float32)

    def zb(i, _):
      for j in range(H // L):
        zbuf[i, pl.ds(j * L, L)] = zero16
      return 0
    lax.fori_loop(0, ZR, zb, 0)
    for k in range(NPT // ZR):
      pltpu.sync_copy(zbuf, agg_sh.at[pl.ds(sid * NPT + k * ZR, ZR)])

    pltpu.sync_copy(u_hbm.at[pl.ds(ebase, EPW)], u_t)
    pltpu.sync_copy(v_hbm.at[pl.ds(ebase, EPW)], v_t)
    pltpu.sync_copy(norm_hbm.at[pl.ds(ebase, EPW)], norm_t)
    if with_type_and_degrees:
      t_t, odl, idl = scr["t_t"], scr["odl"], scr["idl"]
      pltpu.sync_copy(t_hbm.at[pl.ds(ebase, EPW)], t_t)
      one16 = jnp.full((L,), 1.0, jnp.float32)

      def zd(i, _):
        odl[pl.ds(i * L, L)] = zero16
        idl[pl.ds(i * L, L)] = zero16
        return 0
      lax.fori_loop(0, N // L, zd, 0)

    plsc.subcore_barrier()

    def gfire(b, c):
      base = c * CH
      for j in range(CH // L):
        off = base + j * L
        uu = u_t[pl.ds(off, L)]
        vv = v_t[pl.ds(off, L)]
        if with_type_and_degrees:
          tt = t_t[pl.ds(off, L)]
          idxg[b][pl.ds(j * L, L)] = tt * N + uu
          plsc.addupdate_scatter(odl, [uu], one16)
          plsc.addupdate_scatter(idl, [vv], one16)
        else:
          idxg[b][pl.ds(j * L, L)] = uu
        idxs[b][pl.ds(j * L, L)] = vv
      pltpu.async_copy(tab_hbm.at[idxg[b]], gdst[b], gsem[b])

    def process(b, c):
      pltpu.make_async_copy(tab_hbm.at[idxg[b]], gdst[b], gsem[b]).wait()

      @plsc.parallel_loop(0, CH // L, 1, unroll=2)
      def _(jj):
        nvec = norm_t[pl.ds(c * CH + jj * L, L)]
        for i2 in range(L):
          bs = jnp.full((L,), nvec[i2], jnp.float32)
          row = jj * L + i2
          if bf16_table:
            for g in range(H // (2 * L)):
              vbf = gdst[b][row, pl.ds(2 * L * g, 2 * L)]
              ea, ob = plsc.unpack(vbf, format=plsc.PackFormat.INTERLEAVED)
              rows[b][row, pl.ds(2 * L * g, L)] = ea * bs
              rows[b][row, pl.ds(2 * L * g + L, L)] = ob * bs
          else:
            for j2 in range(H // L):
              rows[b][row, pl.ds(j2 * L, L)] = (
                  rows[b][row, pl.ds(j2 * L, L)] * bs)

      pltpu.async_copy(rows[b], agg_sh.at[idxs[b]], ssem[b], add=True)

    def swait(b):
      pltpu.make_async_copy(rows[b], agg_sh.at[idxs[b]], ssem[b]).wait()

    assert (NCH - 2) % 3 == 0 and NCH >= 5
    gfire(0, 0)
    gfire(1, 1)

    def loop3(g, _):
      c0 = 3 * g
      for b in range(3):
        c = c0 + b
        process(b, c)

        @pl.when(c >= 1)
        def _():
          swait((b + 2) % 3)
        gfire((b + 2) % 3, c + 2)
      return 0
    lax.fori_loop(0, (NCH - 2) // 3, loop3, 0)
    process(0, NCH - 2)
    swait(2)
    process(1, NCH - 1)
    swait(0)
    swait(1)

    plsc.subcore_barrier()

    pltpu.sync_copy(agg_sh.at[pl.ds(sid * NPT, NPT)], agg_out.at[cid, sid])
    if with_type_and_degrees:
      for b in range(ND):
        pltpu.sync_copy(odl.at[pl.ds(b * DB, DB)], od_out.at[b, wid, 0])
        pltpu.sync_copy(idl.at[pl.ds(b * DB, DB)], id_out.at[b, wid, 0])

  kern = pl.kernel(body, out_type=tuple(out_type), mesh=mesh,
                   scratch_types=types,
                   compiler_params=pltpu.CompilerParams(
                       needs_layout_passes=False,
                       use_tc_tiling_on_sc=False))
  return kern


def _tc_xw(x, coeff, basis, lw, bias, xw_dtype=jnp.float32):
  N, G = x.shape
  R, NB = coeff.shape
  H = basis.shape[2]
  BN = 2000
  NBLK = N // BN

  def body(x_ref, c_ref, ba_ref, lw_ref, b_ref, xw_ref, xl_ref, w_scr):
    i = pl.program_id(0)
    r = pl.program_id(1)

    @pl.when((i == 0) & (r == 0))
    def _():
      for rr in range(R):
        acc = jnp.zeros((G, H), jnp.float32)
        for bb in range(NB):
          acc = acc + c_ref[rr, bb] * ba_ref[bb]
        w_scr[rr] = acc

    xb = x_ref[...]
    xw_ref[...] = jnp.dot(
        xb, w_scr[r], preferred_element_type=jnp.float32).astype(xw_dtype)

    @pl.when(r == 0)
    def _():
      xl_ref[...] = jnp.dot(xb, lw_ref[...],
                            preferred_element_type=jnp.float32) + b_ref[...]

  return pl.pallas_call(
      body,
      grid=(NBLK, R),
      in_specs=[
          pl.BlockSpec((BN, G), lambda i, r: (i, 0)),
          pl.BlockSpec(memory_space=pltpu.SMEM),
          pl.BlockSpec((NB, G, H), lambda i, r: (0, 0, 0)),
          pl.BlockSpec((G, H), lambda i, r: (0, 0)),
          pl.BlockSpec((1, H), lambda i, r: (0, 0)),
      ],
      out_specs=[
          pl.BlockSpec((BN, H), lambda i, r: (r * NBLK + i, 0)),
          pl.BlockSpec((BN, H), lambda i, r: (i, 0)),
      ],
      out_shape=[
          jax.ShapeDtypeStruct((R * N, H), xw_dtype),
          jax.ShapeDtypeStruct((N, H), jnp.float32),
      ],
      scratch_shapes=[pltpu.VMEM((R, G, H), jnp.float32)],
  )(x, coeff, basis, lw, bias)


def _tc_mid(aggp, xloop, odp):
  _, N, H = aggp.shape
  NWp = odp.shape[1]
  BN = odp.shape[2]
  NBLK = N // BN

  def body(a_ref, xl_ref, od_ref, o_ref):
    h = a_ref[0] + a_ref[1] + xl_ref[...]
    od = jnp.clip(jnp.sum(od_ref[0], axis=0), 1.0, None)
    o_ref[...] = h * lax.rsqrt(od)[:, None]

  return pl.pallas_call(
      body,
      grid=(NBLK,),
      in_specs=[
          pl.BlockSpec((2, BN, H), lambda i: (0, i, 0)),
          pl.BlockSpec((BN, H), lambda i: (i, 0)),
          pl.BlockSpec((1, NWp, BN), lambda i: (i, 0, 0)),
      ],
      out_specs=pl.BlockSpec((BN, H), lambda i: (i, 0)),
      out_shape=jax.ShapeDtypeStruct((N, H), jnp.float32),
  )(aggp, xloop, odp)


def _tc_out(agg2p, idp, W2, b2):
  _, N, H = agg2p.shape
  NWp = idp.shape[1]
  BN = idp.shape[2]
  H2 = W2.shape[1]
  NBLK = N // BN

  def body(a_ref, id_ref, w_ref, b_ref, o_ref):
    h2 = a_ref[0] + a_ref[1]
    idg = jnp.clip(jnp.sum(id_ref[0], axis=0), 1.0, None)
    h2 = h2 * lax.rsqrt(idg)[:, None]
    o_ref[...] = jnp.dot(h2, w_ref[...],
                         preferred_element_type=jnp.float32) + b_ref[...]

  return pl.pallas_call(
      body,
      grid=(NBLK,),
      in_specs=[
          pl.BlockSpec((2, BN, H), lambda i: (0, i, 0)),
          pl.BlockSpec((1, NWp, BN), lambda i: (i, 0, 0)),
          pl.BlockSpec((H, H2), lambda i: (0, 0)),
          pl.BlockSpec((1, H2), lambda i: (0, 0)),
      ],
      out_specs=pl.BlockSpec((BN, H2), lambda i: (i, 0)),
      out_shape=jax.ShapeDtypeStruct((N, H2), jnp.float32),
  )(agg2p, idp, W2, b2)


def kernel(node_features, edge_index, edge_norm, edge_type, basis, coeff,
           loop_weight, bias1, W2, b2):
  N, G = node_features.shape
  NB, _, H1 = basis.shape
  R = coeff.shape[0]
  H2 = W2.shape[1]

  u = edge_index[0]
  v = edge_index[1]
  E = u.shape[0]

  perm = []
  for g in range(H1 // 32):
    for j in range(16):
      perm.extend((32 * g + j, 32 * g + 16 + j))
  basis_p = basis[:, :, jnp.array(perm, dtype=jnp.int32)]
  xw, xloop = _tc_xw(node_features, coeff, basis_p, loop_weight,
                     bias1.reshape(1, H1), xw_dtype=jnp.bfloat16)

  sc1 = _sc_edge_pass(E, N, H1, with_type_and_degrees=True, bf16_table=True)
  aggp, odp, idp = sc1(u, v, edge_type, edge_norm, xw)
  aggp = aggp.reshape(NC, N, H1)
  odp = odp.reshape(odp.shape[0], NW, odp.shape[3])
  idp = idp.reshape(idp.shape[0], NW, idp.shape[3])

  h_src = _tc_mid(aggp, xloop, odp)

  sc2 = _sc_edge_pass(E, N, H1, with_type_and_degrees=False)
  (agg2p,) = sc2(u, v, edge_norm, h_src)
  agg2p = agg2p.reshape(NC, N, H1)

  return _tc_out(agg2p, idp, W2, b2.reshape(1, H2))


# speedup vs baseline: 1.0911x; 1.0911x over previous
import functools

import jax
import jax.numpy as jnp
from jax import lax
from jax.experimental import pallas as pl
from jax.experimental.pallas import tpu as pltpu
from jax.experimental.pallas import tpu_sc as plsc

NC = 2
NS = 16
L = 16
NW = NC * NS

CH = 80


def _sc_edge_pass(E, N, H, with_type_and_degrees):
  EPW = E // NW
  NCH = EPW // CH
  NPT = N // NS
  ZR = 25
  DB = 1000
  ND = N // DB
  assert E % NW == 0 and EPW % CH == 0 and N % NS == 0 and NPT % ZR == 0
  assert N % DB == 0

  mesh = plsc.VectorSubcoreMesh(core_axis_name="c", subcore_axis_name="s")

  out_type = [jax.ShapeDtypeStruct((NC, NS, NPT, H), jnp.float32)]
  if with_type_and_degrees:
    out_type += [jax.ShapeDtypeStruct((ND, NW, 1, DB), jnp.float32),
                 jax.ShapeDtypeStruct((ND, NW, 1, DB), jnp.float32)]

  scratch = dict(
      u_t=pltpu.VMEM((EPW,), jnp.int32),
      v_t=pltpu.VMEM((EPW,), jnp.int32),
      norm_t=pltpu.VMEM((EPW,), jnp.float32),
      idxg0=pltpu.VMEM((CH,), jnp.int32),
      idxg1=pltpu.VMEM((CH,), jnp.int32),
      idxg2=pltpu.VMEM((CH,), jnp.int32),
      idxs0=pltpu.VMEM((CH,), jnp.int32),
      idxs1=pltpu.VMEM((CH,), jnp.int32),
      idxs2=pltpu.VMEM((CH,), jnp.int32),
      rows0=pltpu.VMEM((CH, H), jnp.float32),
      rows1=pltpu.VMEM((CH, H), jnp.float32),
      rows2=pltpu.VMEM((CH, H), jnp.float32),
      zbuf=pltpu.VMEM((ZR, H), jnp.float32),
      agg_sh=pltpu.VMEM_SHARED((N, H), jnp.float32),
      gsem0=pltpu.SemaphoreType.DMA,
      gsem1=pltpu.SemaphoreType.DMA,
      gsem2=pltpu.SemaphoreType.DMA,
      ssem0=pltpu.SemaphoreType.DMA,
      ssem1=pltpu.SemaphoreType.DMA,
      ssem2=pltpu.SemaphoreType.DMA,
  )
  if with_type_and_degrees:
    scratch.update(
        t_t=pltpu.VMEM((EPW,), jnp.int32),
        odl=pltpu.VMEM((N,), jnp.float32),
        idl=pltpu.VMEM((N,), jnp.float32),
    )

  names = list(scratch.keys())
  types = [scratch[k] for k in names]

  def body(*refs):
    n_in = 5 if with_type_and_degrees else 4
    n_out = len(out_type)
    ins = refs[:n_in]
    outs = refs[n_in:n_in + n_out]
    scr = dict(zip(names, refs[n_in + n_out:]))
    if with_type_and_degrees:
      u_hbm, v_hbm, t_hbm, norm_hbm, tab_hbm = ins
      agg_out, od_out, id_out = outs
    else:
      u_hbm, v_hbm, norm_hbm, tab_hbm = ins
      (agg_out,) = outs
    u_t, v_t, norm_t = scr["u_t"], scr["v_t"], scr["norm_t"]
    idxg = [scr["idxg0"], scr["idxg1"], scr["idxg2"]]
    idxs = [scr["idxs0"], scr["idxs1"], scr["idxs2"]]
    rows = [scr["rows0"], scr["rows1"], scr["rows2"]]
    gsem = [scr["gsem0"], scr["gsem1"], scr["gsem2"]]
    ssem = [scr["ssem0"], scr["ssem1"], scr["ssem2"]]
    zbuf, agg_sh = scr["zbuf"], scr["agg_sh"]

    cid = lax.axis_index("c")
    sid = lax.axis_index("s")
    wid = sid * NC + cid
    ebase = wid * EPW

    zero16 = jnp.zeros((L,), jnp.float32)

    pend = [
        pltpu.async_copy(u_hbm.at[pl.ds(ebase, EPW)], u_t, gsem[0]),
        pltpu.async_copy(v_hbm.at[pl.ds(ebase, EPW)], v_t, gsem[1]),
        pltpu.async_copy(norm_hbm.at[pl.ds(ebase, EPW)], norm_t, gsem[2]),
    ]
    if with_type_and_degrees:
      t_t, odl, idl = scr["t_t"], scr["odl"], scr["idl"]
      pend.append(pltpu.async_copy(t_hbm.at[pl.ds(ebase, EPW)], t_t, ssem[0]))
      one16 = jnp.full((L,), 1.0, jnp.float32)

    @plsc.parallel_loop(0, ZR, 1, unroll=2)
    def _(i):
      for j in range(H // L):
        zbuf[i, pl.ds(j * L, L)] = zero16

    pend.extend(
        pltpu.async_copy(zbuf, agg_sh.at[pl.ds(sid * NPT + k * ZR, ZR)],
                         ssem[1])
        for k in range(NPT // ZR))

    if with_type_and_degrees:
      @plsc.parallel_loop(0, N // L, 1, unroll=4)
      def _(i):
        odl[pl.ds(i * L, L)] = zero16
        idl[pl.ds(i * L, L)] = zero16

    for d in pend:
      d.wait()
    plsc.subcore_barrier()

    def gfire(b, c):
      base = c * CH
      for j in range(CH // L):
        off = base + j * L
        uu = u_t[pl.ds(off, L)]
        vv = v_t[pl.ds(off, L)]
        if with_type_and_degrees:
          tt = t_t[pl.ds(off, L)]
          idxg[b][pl.ds(j * L, L)] = tt * N + uu
          plsc.addupdate_scatter(odl, [uu], one16)
          plsc.addupdate_scatter(idl, [vv], one16)
        else:
          idxg[b][pl.ds(j * L, L)] = uu
        idxs[b][pl.ds(j * L, L)] = vv
      pltpu.async_copy(tab_hbm.at[idxg[b]], rows[b], gsem[b])

    def process(b, c):
      pltpu.make_async_copy(tab_hbm.at[idxg[b]], rows[b], gsem[b]).wait()

      @plsc.parallel_loop(0, CH // L, 1, unroll=2)
      def _(jj):
        nvec = norm_t[pl.ds(c * CH + jj * L, L)]
        for i2 in range(L):
          bs = jnp.full((L,), nvec[i2], jnp.float32)
          row = jj * L + i2
          for j2 in range(H // L):
            rows[b][row, pl.ds(j2 * L, L)] = (
                rows[b][row, pl.ds(j2 * L, L)] * bs)

      pltpu.async_copy(rows[b], agg_sh.at[idxs[b]], ssem[b], add=True)

    def swait(b):
      pltpu.make_async_copy(rows[b], agg_sh.at[idxs[b]], ssem[b]).wait()

    assert (NCH - 2) % 3 == 0 and NCH >= 5
    gfire(0, 0)
    gfire(1, 1)

    def loop3(g, _):
      c0 = 3 * g
      for b in range(3):
        c = c0 + b
        process(b, c)

        @pl.when(c >= 1)
        def _():
          swait((b + 2) % 3)
        gfire((b + 2) % 3, c + 2)
      return 0
    lax.fori_loop(0, (NCH - 2) // 3, loop3, 0)
    process(0, NCH - 2)
    swait(2)
    process(1, NCH - 1)
    swait(0)
    swait(1)

    plsc.subcore_barrier()

    fin = [pltpu.async_copy(agg_sh.at[pl.ds(sid * NPT, NPT)],
                            agg_out.at[cid, sid], gsem[0])]
    if with_type_and_degrees:
      for b in range(ND):
        fin.append(pltpu.async_copy(odl.at[pl.ds(b * DB, DB)],
                                    od_out.at[b, wid, 0], gsem[1]))
        fin.append(pltpu.async_copy(idl.at[pl.ds(b * DB, DB)],
                                    id_out.at[b, wid, 0], gsem[2]))
    for d in fin:
      d.wait()

  kern = pl.kernel(body, out_type=tuple(out_type), mesh=mesh,
                   scratch_types=types,
                   compiler_params=pltpu.CompilerParams(
                       needs_layout_passes=False,
                       use_tc_tiling_on_sc=False))
  return kern


def _tc_xw(x, coeff, basis, lw, bias, xw_dtype=jnp.float32):
  N, G = x.shape
  R, NB = coeff.shape
  H = basis.shape[2]
  BN = 2000
  NBLK = N // BN

  def body(x_ref, c_ref, ba_ref, lw_ref, b_ref, xw_ref, xl_ref, w_scr):
    i = pl.program_id(0)
    r = pl.program_id(1)

    @pl.when((i == 0) & (r == 0))
    def _():
      for rr in range(R):
        acc = jnp.zeros((G, H), jnp.float32)
        for bb in range(NB):
          acc = acc + c_ref[rr, bb] * ba_ref[bb]
        w_scr[rr] = acc

    xb = x_ref[...]
    xw_ref[...] = jnp.dot(
        xb, w_scr[r], preferred_element_type=jnp.float32).astype(xw_dtype)

    @pl.when(r == 0)
    def _():
      xl_ref[...] = jnp.dot(xb, lw_ref[...],
                            preferred_element_type=jnp.float32) + b_ref[...]

  return pl.pallas_call(
      body,
      grid=(NBLK, R),
      in_specs=[
          pl.BlockSpec((BN, G), lambda i, r: (i, 0)),
          pl.BlockSpec(memory_space=pltpu.SMEM),
          pl.BlockSpec((NB, G, H), lambda i, r: (0, 0, 0)),
          pl.BlockSpec((G, H), lambda i, r: (0, 0)),
          pl.BlockSpec((1, H), lambda i, r: (0, 0)),
      ],
      out_specs=[
          pl.BlockSpec((BN, H), lambda i, r: (r * NBLK + i, 0)),
          pl.BlockSpec((BN, H), lambda i, r: (i, 0)),
      ],
      out_shape=[
          jax.ShapeDtypeStruct((R * N, H), xw_dtype),
          jax.ShapeDtypeStruct((N, H), jnp.float32),
      ],
      scratch_shapes=[pltpu.VMEM((R, G, H), jnp.float32)],
  )(x, coeff, basis, lw, bias)


def _tc_mid(aggp, xloop, odp):
  _, N, H = aggp.shape
  NWp = odp.shape[1]
  BN = odp.shape[2]
  NBLK = N // BN

  def body(a_ref, xl_ref, od_ref, o_ref):
    h = a_ref[0] + a_ref[1] + xl_ref[...]
    od = jnp.clip(jnp.sum(od_ref[0], axis=0), 1.0, None)
    o_ref[...] = h * lax.rsqrt(od)[:, None]

  return pl.pallas_call(
      body,
      grid=(NBLK,),
      in_specs=[
          pl.BlockSpec((2, BN, H), lambda i: (0, i, 0)),
          pl.BlockSpec((BN, H), lambda i: (i, 0)),
          pl.BlockSpec((1, NWp, BN), lambda i: (i, 0, 0)),
      ],
      out_specs=pl.BlockSpec((BN, H), lambda i: (i, 0)),
      out_shape=jax.ShapeDtypeStruct((N, H), jnp.float32),
  )(aggp, xloop, odp)


def _tc_out(agg2p, idp, W2, b2):
  _, N, H = agg2p.shape
  NWp = idp.shape[1]
  BN = idp.shape[2]
  H2 = W2.shape[1]
  NBLK = N // BN

  def body(a_ref, id_ref, w_ref, b_ref, o_ref):
    h2 = a_ref[0] + a_ref[1]
    idg = jnp.clip(jnp.sum(id_ref[0], axis=0), 1.0, None)
    h2 = h2 * lax.rsqrt(idg)[:, None]
    o_ref[...] = jnp.dot(h2, w_ref[...],
                         preferred_element_type=jnp.float32) + b_ref[...]

  return pl.pallas_call(
      body,
      grid=(NBLK,),
      in_specs=[
          pl.BlockSpec((2, BN, H), lambda i: (0, i, 0)),
          pl.BlockSpec((1, NWp, BN), lambda i: (i, 0, 0)),
          pl.BlockSpec((H, H2), lambda i: (0, 0)),
          pl.BlockSpec((1, H2), lambda i: (0, 0)),
      ],
      out_specs=pl.BlockSpec((BN, H2), lambda i: (i, 0)),
      out_shape=jax.ShapeDtypeStruct((N, H2), jnp.float32),
  )(agg2p, idp, W2, b2)


def kernel(node_features, edge_index, edge_norm, edge_type, basis, coeff,
           loop_weight, bias1, W2, b2):
  N, G = node_features.shape
  NB, _, H1 = basis.shape
  R = coeff.shape[0]
  H2 = W2.shape[1]

  u = edge_index[0]
  v = edge_index[1]
  E = u.shape[0]

  xw, xloop = _tc_xw(node_features, coeff, basis, loop_weight,
                     bias1.reshape(1, H1))

  sc1 = _sc_edge_pass(E, N, H1, with_type_and_degrees=True)
  aggp, odp, idp = sc1(u, v, edge_type, edge_norm, xw)
  aggp = aggp.reshape(NC, N, H1)
  odp = odp.reshape(odp.shape[0], NW, odp.shape[3])
  idp = idp.reshape(idp.shape[0], NW, idp.shape[3])

  h_src = _tc_mid(aggp, xloop, odp)

  sc2 = _sc_edge_pass(E, N, H1, with_type_and_degrees=False)
  (agg2p,) = sc2(u, v, edge_norm, h_src)
  agg2p = agg2p.reshape(NC, N, H1)

  return _tc_out(agg2p, idp, W2, b2.reshape(1, H2))


# speedup vs baseline: 1.0912x; 1.0001x over previous
import jax
import jax.numpy as jnp
from jax import lax
from jax.experimental import pallas as pl
from jax.experimental.pallas import tpu as pltpu
from jax.experimental.pallas import tpu_sc as plsc

NC = 2
NS = 16
L = 16
NW = NC * NS

CH = 80


def _sc_edge_pass(E, N, H, with_type_and_degrees):
  EPW = E // NW
  NCH = EPW // CH
  NPT = N // NS
  ZR = 25
  DB = 1000
  ND = N // DB
  assert E % NW == 0 and EPW % CH == 0 and N % NS == 0 and NPT % ZR == 0
  assert N % DB == 0

  mesh = plsc.VectorSubcoreMesh(core_axis_name="c", subcore_axis_name="s")

  out_type = [jax.ShapeDtypeStruct((NC, NS, NPT, H), jnp.float32)]
  if with_type_and_degrees:
    out_type += [jax.ShapeDtypeStruct((ND, NW, 1, DB), jnp.float32),
                 jax.ShapeDtypeStruct((ND, NW, 1, DB), jnp.float32)]

  scratch = dict(
      u_t=pltpu.VMEM((EPW,), jnp.int32),
      v_t=pltpu.VMEM((EPW,), jnp.int32),
      norm_t=pltpu.VMEM((EPW,), jnp.float32),
      idxg0=pltpu.VMEM((CH,), jnp.int32),
      idxg1=pltpu.VMEM((CH,), jnp.int32),
      idxg2=pltpu.VMEM((CH,), jnp.int32),
      idxs0=pltpu.VMEM((CH,), jnp.int32),
      idxs1=pltpu.VMEM((CH,), jnp.int32),
      idxs2=pltpu.VMEM((CH,), jnp.int32),
      rows0=pltpu.VMEM((CH, H), jnp.float32),
      rows1=pltpu.VMEM((CH, H), jnp.float32),
      rows2=pltpu.VMEM((CH, H), jnp.float32),
      zbuf=pltpu.VMEM((ZR, H), jnp.float32),
      agg_sh=pltpu.VMEM_SHARED((N, H), jnp.float32),
      gsem0=pltpu.SemaphoreType.DMA,
      gsem1=pltpu.SemaphoreType.DMA,
      gsem2=pltpu.SemaphoreType.DMA,
      ssem0=pltpu.SemaphoreType.DMA,
      ssem1=pltpu.SemaphoreType.DMA,
      ssem2=pltpu.SemaphoreType.DMA,
  )
  if with_type_and_degrees:
    scratch.update(
        t_t=pltpu.VMEM((EPW,), jnp.int32),
        odl=pltpu.VMEM((N,), jnp.float32),
        idl=pltpu.VMEM((N,), jnp.float32),
    )

  names = list(scratch.keys())
  types = [scratch[k] for k in names]

  def body(*refs):
    n_in = 5 if with_type_and_degrees else 4
    n_out = len(out_type)
    ins = refs[:n_in]
    outs = refs[n_in:n_in + n_out]
    scr = dict(zip(names, refs[n_in + n_out:]))
    if with_type_and_degrees:
      u_hbm, v_hbm, t_hbm, norm_hbm, tab_hbm = ins
      agg_out, od_out, id_out = outs
    else:
      u_hbm, v_hbm, norm_hbm, tab_hbm = ins
      (agg_out,) = outs
    u_t, v_t, norm_t = scr["u_t"], scr["v_t"], scr["norm_t"]
    idxg = [scr["idxg0"], scr["idxg1"], scr["idxg2"]]
    idxs = [scr["idxs0"], scr["idxs1"], scr["idxs2"]]
    rows = [scr["rows0"], scr["rows1"], scr["rows2"]]
    gsem = [scr["gsem0"], scr["gsem1"], scr["gsem2"]]
    ssem = [scr["ssem0"], scr["ssem1"], scr["ssem2"]]
    zbuf, agg_sh = scr["zbuf"], scr["agg_sh"]

    cid = lax.axis_index("c")
    sid = lax.axis_index("s")
    wid = sid * NC + cid
    ebase = wid * EPW

    zero16 = jnp.zeros((L,), jnp.float32)

    pend = [
        pltpu.async_copy(u_hbm.at[pl.ds(ebase, EPW)], u_t, gsem[0]),
        pltpu.async_copy(v_hbm.at[pl.ds(ebase, EPW)], v_t, gsem[1]),
        pltpu.async_copy(norm_hbm.at[pl.ds(ebase, EPW)], norm_t, gsem[2]),
    ]
    if with_type_and_degrees:
      t_t, odl, idl = scr["t_t"], scr["odl"], scr["idl"]
      pend.append(pltpu.async_copy(t_hbm.at[pl.ds(ebase, EPW)], t_t, ssem[0]))
      one16 = jnp.full((L,), 1.0, jnp.float32)

    @plsc.parallel_loop(0, ZR, 1, unroll=2)
    def _(i):
      for j in range(H // L):
        zbuf[i, pl.ds(j * L, L)] = zero16

    pend.extend(
        pltpu.async_copy(zbuf, agg_sh.at[pl.ds(sid * NPT + k * ZR, ZR)],
                         ssem[1])
        for k in range(NPT // ZR))

    if with_type_and_degrees:
      @plsc.parallel_loop(0, N // L, 1, unroll=4)
      def _(i):
        odl[pl.ds(i * L, L)] = zero16
        idl[pl.ds(i * L, L)] = zero16

    for d in pend:
      d.wait()
    plsc.subcore_barrier()

    def gfire(b, c):
      base = c * CH
      for j in range(CH // L):
        off = base + j * L
        uu = u_t[pl.ds(off, L)]
        vv = v_t[pl.ds(off, L)]
        if with_type_and_degrees:
          tt = t_t[pl.ds(off, L)]
          idxg[b][pl.ds(j * L, L)] = tt * N + uu
          plsc.addupdate_scatter(odl, [uu], one16)
          plsc.addupdate_scatter(idl, [vv], one16)
        else:
          idxg[b][pl.ds(j * L, L)] = uu
        idxs[b][pl.ds(j * L, L)] = vv
      pltpu.async_copy(tab_hbm.at[idxg[b]], rows[b], gsem[b])

    def process(b, c):
      pltpu.make_async_copy(tab_hbm.at[idxg[b]], rows[b], gsem[b]).wait()

      @plsc.parallel_loop(0, CH // L, 1, unroll=2)
      def _(jj):
        nvec = norm_t[pl.ds(c * CH + jj * L, L)]
        for i2 in range(L):
          bs = jnp.full((L,), nvec[i2], jnp.float32)
          row = jj * L + i2
          for j2 in range(H // L):
            rows[b][row, pl.ds(j2 * L, L)] = (
                rows[b][row, pl.ds(j2 * L, L)] * bs)

      pltpu.async_copy(rows[b], agg_sh.at[idxs[b]], ssem[b], add=True)

    def swait(b):
      pltpu.make_async_copy(rows[b], agg_sh.at[idxs[b]], ssem[b]).wait()

    assert (NCH - 2) % 3 == 0 and NCH >= 5
    gfire(0, 0)
    gfire(1, 1)

    def loop3(g, _):
      c0 = 3 * g
      for b in range(3):
        c = c0 + b
        process(b, c)

        @pl.when(c >= 1)
        def _():
          swait((b + 2) % 3)
        gfire((b + 2) % 3, c + 2)
      return 0
    lax.fori_loop(0, (NCH - 2) // 3, loop3, 0)
    process(0, NCH - 2)
    swait(2)
    process(1, NCH - 1)
    swait(0)
    swait(1)

    plsc.subcore_barrier()

    fin = [pltpu.async_copy(agg_sh.at[pl.ds(sid * NPT, NPT)],
                            agg_out.at[cid, sid], gsem[0])]
    if with_type_and_degrees:
      for b in range(ND):
        fin.append(pltpu.async_copy(odl.at[pl.ds(b * DB, DB)],
                                    od_out.at[b, wid, 0], gsem[1]))
        fin.append(pltpu.async_copy(idl.at[pl.ds(b * DB, DB)],
                                    id_out.at[b, wid, 0], gsem[2]))
    for d in fin:
      d.wait()

  kern = pl.kernel(body, out_type=tuple(out_type), mesh=mesh,
                   scratch_types=types,
                   compiler_params=pltpu.CompilerParams(
                       needs_layout_passes=False,
                       use_tc_tiling_on_sc=False))
  return kern


def _tc_xw(x, coeff, basis, lw, bias, xw_dtype=jnp.float32):
  N, G = x.shape
  R, NB = coeff.shape
  H = basis.shape[2]
  BN = 2000
  NBLK = N // BN

  def body(x_ref, c_ref, ba_ref, lw_ref, b_ref, xw_ref, xl_ref, w_scr):
    i = pl.program_id(0)
    r = pl.program_id(1)

    @pl.when((i == 0) & (r == 0))
    def _():
      for rr in range(R):
        acc = jnp.zeros((G, H), jnp.float32)
        for bb in range(NB):
          acc = acc + c_ref[rr, bb] * ba_ref[bb]
        w_scr[rr] = acc

    xb = x_ref[...]
    xw_ref[...] = jnp.dot(
        xb, w_scr[r], preferred_element_type=jnp.float32).astype(xw_dtype)

    @pl.when(r == 0)
    def _():
      xl_ref[...] = jnp.dot(xb, lw_ref[...],
                            preferred_element_type=jnp.float32) + b_ref[...]

  return pl.pallas_call(
      body,
      grid=(NBLK, R),
      in_specs=[
          pl.BlockSpec((BN, G), lambda i, r: (i, 0)),
          pl.BlockSpec(memory_space=pltpu.SMEM),
          pl.BlockSpec((NB, G, H), lambda i, r: (0, 0, 0)),
          pl.BlockSpec((G, H), lambda i, r: (0, 0)),
          pl.BlockSpec((1, H), lambda i, r: (0, 0)),
      ],
      out_specs=[
          pl.BlockSpec((BN, H), lambda i, r: (r * NBLK + i, 0)),
          pl.BlockSpec((BN, H), lambda i, r: (i, 0)),
      ],
      out_shape=[
          jax.ShapeDtypeStruct((R * N, H), xw_dtype),
          jax.ShapeDtypeStruct((N, H), jnp.float32),
      ],
      scratch_shapes=[pltpu.VMEM((R, G, H), jnp.float32)],
  )(x, coeff, basis, lw, bias)


def _tc_mid(aggp, xloop, odp):
  _, N, H = aggp.shape
  NWp = odp.shape[1]
  BN = odp.shape[2]
  NBLK = N // BN

  def body(a_ref, xl_ref, od_ref, o_ref):
    h = a_ref[0] + a_ref[1] + xl_ref[...]
    od = jnp.clip(jnp.sum(od_ref[0], axis=0), 1.0, None)
    o_ref[...] = h * lax.rsqrt(od)[:, None]

  return pl.pallas_call(
      body,
      grid=(NBLK,),
      in_specs=[
          pl.BlockSpec((2, BN, H), lambda i: (0, i, 0)),
          pl.BlockSpec((BN, H), lambda i: (i, 0)),
          pl.BlockSpec((1, NWp, BN), lambda i: (i, 0, 0)),
      ],
      out_specs=pl.BlockSpec((BN, H), lambda i: (i, 0)),
      out_shape=jax.ShapeDtypeStruct((N, H), jnp.float32),
  )(aggp, xloop, odp)


def _tc_out(agg2p, idp, W2, b2):
  _, N, H = agg2p.shape
  NWp = idp.shape[1]
  BN = idp.shape[2]
  H2 = W2.shape[1]
  NBLK = N // BN

  def body(a_ref, id_ref, w_ref, b_ref, o_ref):
    h2 = a_ref[0] + a_ref[1]
    idg = jnp.clip(jnp.sum(id_ref[0], axis=0), 1.0, None)
    h2 = h2 * lax.rsqrt(idg)[:, None]
    o_ref[...] = jnp.dot(h2, w_ref[...],
                         preferred_element_type=jnp.float32) + b_ref[...]

  return pl.pallas_call(
      body,
      grid=(NBLK,),
      in_specs=[
          pl.BlockSpec((2, BN, H), lambda i: (0, i, 0)),
          pl.BlockSpec((1, NWp, BN), lambda i: (i, 0, 0)),
          pl.BlockSpec((H, H2), lambda i: (0, 0)),
          pl.BlockSpec((1, H2), lambda i: (0, 0)),
      ],
      out_specs=pl.BlockSpec((BN, H2), lambda i: (i, 0)),
      out_shape=jax.ShapeDtypeStruct((N, H2), jnp.float32),
  )(agg2p, idp, W2, b2)


def kernel(node_features, edge_index, edge_norm, edge_type, basis, coeff,
           loop_weight, bias1, W2, b2):
  N, G = node_features.shape
  NB, _, H1 = basis.shape
  R = coeff.shape[0]
  H2 = W2.shape[1]

  u = edge_index[0]
  v = edge_index[1]
  E = u.shape[0]

  xw, xloop = _tc_xw(node_features, coeff, basis, loop_weight,
                     bias1.reshape(1, H1))

  sc1 = _sc_edge_pass(E, N, H1, with_type_and_degrees=True)
  aggp, odp, idp = sc1(u, v, edge_type, edge_norm, xw)
  aggp = aggp.reshape(NC, N, H1)
  odp = odp.reshape(odp.shape[0], NW, odp.shape[3])
  idp = idp.reshape(idp.shape[0], NW, idp.shape[3])

  h_src = _tc_mid(aggp, xloop, odp)

  sc2 = _sc_edge_pass(E, N, H1, with_type_and_degrees=False)
  (agg2p,) = sc2(u, v, edge_norm, h_src)
  agg2p = agg2p.reshape(NC, N, H1)

  return _tc_out(agg2p, idp, W2, b2.reshape(1, H2))
